# initial kernel scaffold (unmeasured)
import jax
import jax.numpy as jnp
from jax import lax
from jax.experimental import pallas as pl
from jax.experimental.pallas import tpu as pltpu

N_DEV = 4
K_TAPS = 4
HALO = K_TAPS - 1


def kernel(x, k):
    B, S, C = x.shape

    def body(x_ref, k_ref, out_ref, send_buf, halo_buf, send_sem, recv_sem):
        my_pos = lax.axis_index("i")
        right = lax.rem(my_pos + 1, N_DEV)

        send_buf[...] = x_ref[:, S - HALO :, :]
        rdma = pltpu.make_async_remote_copy(
            src_ref=send_buf,
            dst_ref=halo_buf,
            send_sem=send_sem,
            recv_sem=recv_sem,
            device_id=(right,),
            device_id_type=pl.DeviceIdType.MESH,
        )
        rdma.start()
        rdma.wait()

        halo = halo_buf[...]
        halo = jnp.where(my_pos == 0, jnp.zeros_like(halo), halo)

        kv = k_ref[...].astype(jnp.bfloat16)
        for b in range(B):
            xb = x_ref[b].astype(jnp.bfloat16)
            hb = halo[b].astype(jnp.bfloat16)
            pad = jnp.concatenate([hb, xb], axis=0)
            acc = jnp.zeros((S, C), jnp.float32)
            for t in range(K_TAPS):
                acc += (pad[t : t + S, :] * kv[t][None, :]).astype(jnp.float32)
            out_ref[b] = acc / (1.0 + jnp.exp(-acc))

    return pl.pallas_call(
        body,
        out_shape=jax.ShapeDtypeStruct((B, S, C), jnp.float32),
        in_specs=[
            pl.BlockSpec(memory_space=pltpu.VMEM),
            pl.BlockSpec(memory_space=pltpu.VMEM),
        ],
        out_specs=pl.BlockSpec(memory_space=pltpu.VMEM),
        scratch_shapes=[
            pltpu.VMEM((B, HALO, C), jnp.float32),
            pltpu.VMEM((B, HALO, C), jnp.float32),
            pltpu.SemaphoreType.DMA,
            pltpu.SemaphoreType.DMA,
        ],
        compiler_params=pltpu.CompilerParams(has_side_effects=True),
    )(x, k)


# baseline (device time: 44340 ns/iter reference)
import jax
import jax.numpy as jnp
from jax import lax
from jax.experimental import pallas as pl
from jax.experimental.pallas import tpu as pltpu

N_DEV = 4
K_TAPS = 4
HALO = K_TAPS - 1
T_CHUNK = 512


def _halo_body(tail_ref, halo_ref, send_sem, recv_sem):
    my_pos = lax.axis_index("i")
    right = lax.rem(my_pos + 1, N_DEV)
    rdma = pltpu.make_async_remote_copy(
        src_ref=tail_ref,
        dst_ref=halo_ref,
        send_sem=send_sem,
        recv_sem=recv_sem,
        device_id=(right,),
        device_id_type=pl.DeviceIdType.MESH,
    )
    rdma.start()
    rdma.wait()

    @pl.when(my_pos == 0)
    def _():
        halo_ref[...] = jnp.zeros_like(halo_ref)


def _conv_body(x_ref, halo_ref, k_ref, out_ref):
    S = x_ref.shape[1]
    kv = k_ref[...].astype(jnp.bfloat16)
    halo = halo_ref[0].astype(jnp.bfloat16)
    for c in range(S // T_CHUNK):
        lo = c * T_CHUNK
        if c == 0:
            tail = halo
        else:
            tail = x_ref[0, lo - HALO : lo, :].astype(jnp.bfloat16)
        chunk = x_ref[0, lo : lo + T_CHUNK, :].astype(jnp.bfloat16)
        pad = jnp.concatenate([tail, chunk], axis=0)
        acc = jnp.zeros((T_CHUNK, pad.shape[1]), jnp.float32)
        for t in range(K_TAPS):
            acc += (pad[t : t + T_CHUNK, :] * kv[t][None, :]).astype(jnp.float32)
        out_ref[0, lo : lo + T_CHUNK, :] = acc / (1.0 + jnp.exp(-acc))


def kernel(x, k):
    B, S, C = x.shape

    tail = x[:, S - HALO :, :]
    halo = pl.pallas_call(
        _halo_body,
        out_shape=jax.ShapeDtypeStruct((B, HALO, C), x.dtype),
        in_specs=[pl.BlockSpec(memory_space=pltpu.VMEM)],
        out_specs=pl.BlockSpec(memory_space=pltpu.VMEM),
        scratch_shapes=[
            pltpu.SemaphoreType.DMA,
            pltpu.SemaphoreType.DMA,
        ],
        compiler_params=pltpu.CompilerParams(has_side_effects=True),
    )(tail)

    return pl.pallas_call(
        _conv_body,
        grid=(B,),
        out_shape=jax.ShapeDtypeStruct((B, S, C), jnp.float32),
        in_specs=[
            pl.BlockSpec((1, S, C), lambda b: (b, 0, 0)),
            pl.BlockSpec((1, HALO, C), lambda b: (b, 0, 0)),
            pl.BlockSpec((K_TAPS, C), lambda b: (0, 0)),
        ],
        out_specs=pl.BlockSpec((1, S, C), lambda b: (b, 0, 0)),
        compiler_params=pltpu.CompilerParams(
            vmem_limit_bytes=56 * 1024 * 1024
        ),
    )(x, halo, k)
